# baseline (speedup 1.0000x reference)
import functools

import jax
import jax.numpy as jnp
from jax.experimental import pallas as pl
from jax.experimental.pallas import tpu as pltpu
from jax.experimental.pallas import tpu_sc as plsc

_NC = 2
_NS = 16
_NW = _NC * _NS
_L = 16
_CH = 64


def _sc_gather_u(user_ids, U_gmf, U_mlp):
    B = user_ids.shape[0]
    D = U_gmf.shape[1]
    bpw = B // _NW
    nch = bpw // _CH
    out = jax.ShapeDtypeStruct((B, D), jnp.float32)
    mesh = plsc.VectorSubcoreMesh(core_axis_name="c", subcore_axis_name="s")

    buf_t = pltpu.VMEM((_CH, D), jnp.float32)

    @functools.partial(
        pl.kernel,
        out_type=(out, out),
        mesh=mesh,
        compiler_params=pltpu.CompilerParams(needs_layout_passes=False),
        scratch_types=[
            pltpu.VMEM((bpw,), jnp.int32),
            buf_t, buf_t,
            buf_t, buf_t,
            pltpu.SemaphoreType.DMA,
            pltpu.SemaphoreType.DMA,
        ],
    )
    def gather_kernel(uid_hbm, ug_hbm, um_hbm, oug, oum,
                      uidx_v, b00, b01, b10, b11, gsem, ssem):
        wid = jax.lax.axis_index("s") * _NC + jax.lax.axis_index("c")
        base = wid * bpw
        pltpu.sync_copy(uid_hbm.at[pl.ds(base, bpw)], uidx_v)
        lane = jax.lax.iota(jnp.int32, _L)
        tabs = (ug_hbm, um_hbm)
        outs = (oug, oum)
        bufsets = ((b00, b01), (b10, b11))

        prev_stores = [None, None]
        for c in range(nch):
            bufs = bufsets[c % 2]
            if prev_stores[c % 2] is not None:
                for st in prev_stores[c % 2]:
                    st.wait()

            @pl.loop(0, _CH // _L)
            def _(g):
                off = c * _CH + g * _L
                uvec = uidx_v[pl.ds(off, _L)]
                for l in range(_L):
                    u = jnp.sum(jnp.where(lane == l, uvec, 0))
                    i = pl.ds(g * _L + l, 1)
                    pltpu.async_copy(ug_hbm.at[pl.ds(u, 1)], bufs[0].at[i], gsem)
                    pltpu.async_copy(um_hbm.at[pl.ds(u, 1)], bufs[1].at[i], gsem)

            for t in range(2):
                pltpu.make_async_copy(
                    tabs[t].at[pl.ds(0, _CH)], bufs[t], gsem).wait()
            stores = []
            for t in range(2):
                stores.append(pltpu.async_copy(
                    bufs[t], outs[t].at[pl.ds(base + c * _CH, _CH)], ssem))
            prev_stores[c % 2] = stores

        for sts in prev_stores:
            if sts is not None:
                for st in sts:
                    st.wait()

    return gather_kernel(user_ids, U_gmf, U_mlp)


def _sc_gather_i(item_ids, I_gmf, I_mlp):
    B = item_ids.shape[0]
    D = I_gmf.shape[1]
    bpw = B // _NW
    half = bpw // 2
    out = jax.ShapeDtypeStruct((B, D), jnp.float32)
    mesh = plsc.VectorSubcoreMesh(core_axis_name="c", subcore_axis_name="s")

    @functools.partial(
        pl.kernel,
        out_type=(out, out),
        mesh=mesh,
        compiler_params=pltpu.CompilerParams(use_tc_tiling_on_sc=False),
        scratch_types=[
            pltpu.VMEM((bpw,), jnp.int32),
            pltpu.VMEM((half, D), jnp.float32),
            pltpu.VMEM((half, D), jnp.float32),
            pltpu.SemaphoreType.DMA,
            pltpu.SemaphoreType.DMA,
        ],
    )
    def gather_kernel(iid_hbm, ig_hbm, im_hbm, oig, oim,
                      idx_v, buf0, buf1, gsem, ssem):
        wid = jax.lax.axis_index("s") * _NC + jax.lax.axis_index("c")
        base = wid * bpw
        pltpu.sync_copy(iid_hbm.at[pl.ds(base, bpw)], idx_v)
        srcs = (ig_hbm, ig_hbm, im_hbm, im_hbm)
        outs = (oig, oig, oim, oim)
        offs = (0, half, 0, half)
        prev = None
        prev_store = None
        for k in range(4):
            buf = buf0 if k % 2 == 0 else buf1
            g = pltpu.async_copy(
                srcs[k].at[idx_v.at[pl.ds(offs[k], half)]], buf, gsem)
            if prev is not None:
                pk, pbuf = prev
                if prev_store is not None:
                    prev_store.wait()
                prev_store = pltpu.async_copy(
                    pbuf, outs[pk].at[pl.ds(base + offs[pk], half)], ssem)
            g.wait()
            prev = (k, buf)
        pk, pbuf = prev
        if prev_store is not None:
            prev_store.wait()
        pltpu.sync_copy(pbuf, outs[pk].at[pl.ds(base + offs[pk], half)])

    return gather_kernel(item_ids, I_gmf, I_mlp)


def _mlp_body(ug, ig, um, im, w1u, w1i, b1, w2, b2, w3, b3, wpg, wph, bp, out):
    f32 = jnp.float32
    um_b = um[...].astype(jnp.bfloat16)
    im_b = im[...].astype(jnp.bfloat16)
    h1 = jnp.maximum(
        jnp.dot(um_b, w1u[...], preferred_element_type=f32)
        + jnp.dot(im_b, w1i[...], preferred_element_type=f32)
        + b1[...], 0.0)
    h2 = jnp.maximum(
        jnp.dot(h1.astype(jnp.bfloat16), w2[...], preferred_element_type=f32)
        + b2[...], 0.0)
    h3 = jnp.maximum(
        jnp.dot(h2.astype(jnp.bfloat16), w3[...], preferred_element_type=f32)
        + b3[...], 0.0)
    gmf = ug[...] * ig[...]
    pred = (jnp.sum(gmf * wpg[...], axis=1)
            + jnp.sum(h3 * wph[...], axis=1)
            + bp[...][0, 0])
    out[...] = jax.nn.sigmoid(pred)


def _tc_mlp(ug, ig, um, im, W1, b1, W2, b2, W3, b3, Wp, bp):
    B, D = ug.shape
    H1 = W1.shape[1]
    H2 = W2.shape[1]
    H3 = W3.shape[1]
    BS = 2048
    bf16 = jnp.bfloat16
    w1u = W1[:D].astype(bf16)
    w1i = W1[D:].astype(bf16)
    w2 = W2.astype(bf16)
    w3 = W3.astype(bf16)
    wpg = Wp[:D].reshape(1, D)
    wph = Wp[D:].reshape(1, D)
    b1r = b1.reshape(1, H1)
    b2r = b2.reshape(1, H2)
    b3r = b3.reshape(1, H3)
    bpr = bp.reshape(1, 1)

    emb_spec = pl.BlockSpec((BS, D), lambda i: (i, 0))

    def full(a):
        return pl.BlockSpec(a.shape, lambda i: tuple(0 for _ in a.shape))

    return pl.pallas_call(
        _mlp_body,
        grid=(B // BS,),
        in_specs=[emb_spec, emb_spec, emb_spec, emb_spec,
                  full(w1u), full(w1i), full(b1r), full(w2), full(b2r),
                  full(w3), full(b3r), full(wpg), full(wph), full(bpr)],
        out_specs=pl.BlockSpec((BS,), lambda i: (i,)),
        out_shape=jax.ShapeDtypeStruct((B,), jnp.float32),
    )(ug, ig, um, im, w1u, w1i, b1r, w2, b2r, w3, b3r, wpg, wph, bpr)


def kernel(user_ids, item_ids, U_gmf, I_gmf, U_mlp, I_mlp,
           W1, b1, W2, b2, W3, b3, Wp, bp):
    ug, um = _sc_gather_u(user_ids, U_gmf, U_mlp)
    ig, im = _sc_gather_i(item_ids, I_gmf, I_mlp)
    return _tc_mlp(ug, ig, um, im, W1, b1, W2, b2, W3, b3, Wp, bp)


# speedup vs baseline: 1.0502x; 1.0502x over previous
import functools

import jax
import jax.numpy as jnp
from jax.experimental import pallas as pl
from jax.experimental.pallas import tpu as pltpu
from jax.experimental.pallas import tpu_sc as plsc

_NC = 2
_NS = 16
_NW = _NC * _NS
_L = 16
_CH = 64


def _sc_gather4(user_ids, item_ids, U_gmf, I_gmf, U_mlp, I_mlp):
    B = user_ids.shape[0]
    D = U_gmf.shape[1]
    bpw = B // _NW
    nch = bpw // _CH
    out = jax.ShapeDtypeStruct((B, D), jnp.float32)
    mesh = plsc.VectorSubcoreMesh(core_axis_name="c", subcore_axis_name="s")

    buf_t = pltpu.VMEM((_CH, D), jnp.float32)

    @functools.partial(
        pl.kernel,
        out_type=(out, out, out, out),
        mesh=mesh,
        compiler_params=pltpu.CompilerParams(needs_layout_passes=False),
        scratch_types=[
            pltpu.VMEM((bpw,), jnp.int32),
            pltpu.VMEM((bpw,), jnp.int32),
            buf_t, buf_t, buf_t, buf_t,
            buf_t, buf_t, buf_t, buf_t,
            pltpu.SemaphoreType.DMA,
            pltpu.SemaphoreType.DMA,
        ],
    )
    def gather_kernel(uid_hbm, iid_hbm, ug_hbm, ig_hbm, um_hbm, im_hbm,
                      oug, oig, oum, oim,
                      uidx_v, iidx_v,
                      b00, b01, b02, b03, b10, b11, b12, b13,
                      gsem, ssem):
        wid = jax.lax.axis_index("s") * _NC + jax.lax.axis_index("c")
        base = wid * bpw
        pltpu.sync_copy(uid_hbm.at[pl.ds(base, bpw)], uidx_v)
        pltpu.sync_copy(iid_hbm.at[pl.ds(base, bpw)], iidx_v)
        lane = jax.lax.iota(jnp.int32, _L)
        tabs = (ug_hbm, ig_hbm, um_hbm, im_hbm)
        outs = (oug, oig, oum, oim)
        bufsets = ((b00, b01, b02, b03), (b10, b11, b12, b13))

        prev_stores = [None, None]
        for c in range(nch):
            bufs = bufsets[c % 2]
            if prev_stores[c % 2] is not None:
                for st in prev_stores[c % 2]:
                    st.wait()

            @pl.loop(0, _CH // _L)
            def _(g):
                off = c * _CH + g * _L
                uvec = uidx_v[pl.ds(off, _L)]
                ivec = iidx_v[pl.ds(off, _L)]
                for l in range(_L):
                    u = jnp.sum(jnp.where(lane == l, uvec, 0))
                    v = jnp.sum(jnp.where(lane == l, ivec, 0))
                    i = pl.ds(g * _L + l, 1)
                    pltpu.async_copy(ug_hbm.at[pl.ds(u, 1)], bufs[0].at[i], gsem)
                    pltpu.async_copy(ig_hbm.at[pl.ds(v, 1)], bufs[1].at[i], gsem)
                    pltpu.async_copy(um_hbm.at[pl.ds(u, 1)], bufs[2].at[i], gsem)
                    pltpu.async_copy(im_hbm.at[pl.ds(v, 1)], bufs[3].at[i], gsem)

            for t in range(4):
                pltpu.make_async_copy(
                    tabs[t].at[pl.ds(0, _CH)], bufs[t], gsem).wait()
            stores = []
            for t in range(4):
                stores.append(pltpu.async_copy(
                    bufs[t], outs[t].at[pl.ds(base + c * _CH, _CH)], ssem))
            prev_stores[c % 2] = stores

        for sts in prev_stores:
            if sts is not None:
                for st in sts:
                    st.wait()

    return gather_kernel(user_ids, item_ids, U_gmf, I_gmf, U_mlp, I_mlp)


def _mlp_body(ug, ig, um, im, w1u, w1i, b1, w2, b2, w3, b3, wpg, wph, bp, out):
    f32 = jnp.float32
    um_b = um[...].astype(jnp.bfloat16)
    im_b = im[...].astype(jnp.bfloat16)
    h1 = jnp.maximum(
        jnp.dot(um_b, w1u[...], preferred_element_type=f32)
        + jnp.dot(im_b, w1i[...], preferred_element_type=f32)
        + b1[...], 0.0)
    h2 = jnp.maximum(
        jnp.dot(h1.astype(jnp.bfloat16), w2[...], preferred_element_type=f32)
        + b2[...], 0.0)
    h3 = jnp.maximum(
        jnp.dot(h2.astype(jnp.bfloat16), w3[...], preferred_element_type=f32)
        + b3[...], 0.0)
    gmf = ug[...] * ig[...]
    pred = (jnp.sum(gmf * wpg[...], axis=1)
            + jnp.sum(h3 * wph[...], axis=1)
            + bp[...][0, 0])
    out[...] = jax.nn.sigmoid(pred)


def _tc_mlp(ug, ig, um, im, W1, b1, W2, b2, W3, b3, Wp, bp):
    B, D = ug.shape
    H1 = W1.shape[1]
    H2 = W2.shape[1]
    H3 = W3.shape[1]
    BS = 2048
    bf16 = jnp.bfloat16
    w1u = W1[:D].astype(bf16)
    w1i = W1[D:].astype(bf16)
    w2 = W2.astype(bf16)
    w3 = W3.astype(bf16)
    wpg = Wp[:D].reshape(1, D)
    wph = Wp[D:].reshape(1, D)
    b1r = b1.reshape(1, H1)
    b2r = b2.reshape(1, H2)
    b3r = b3.reshape(1, H3)
    bpr = bp.reshape(1, 1)

    emb_spec = pl.BlockSpec((BS, D), lambda i: (i, 0))

    def full(a):
        return pl.BlockSpec(a.shape, lambda i: tuple(0 for _ in a.shape))

    return pl.pallas_call(
        _mlp_body,
        grid=(B // BS,),
        in_specs=[emb_spec, emb_spec, emb_spec, emb_spec,
                  full(w1u), full(w1i), full(b1r), full(w2), full(b2r),
                  full(w3), full(b3r), full(wpg), full(wph), full(bpr)],
        out_specs=pl.BlockSpec((BS,), lambda i: (i,)),
        out_shape=jax.ShapeDtypeStruct((B,), jnp.float32),
    )(ug, ig, um, im, w1u, w1i, b1r, w2, b2r, w3, b3r, wpg, wph, bpr)


def kernel(user_ids, item_ids, U_gmf, I_gmf, U_mlp, I_mlp,
           W1, b1, W2, b2, W3, b3, Wp, bp):
    ug, ig, um, im = _sc_gather4(user_ids, item_ids, U_gmf, I_gmf, U_mlp, I_mlp)
    return _tc_mlp(ug, ig, um, im, W1, b1, W2, b2, W3, b3, Wp, bp)
